# Initial kernel scaffold; baseline (speedup 1.0000x reference)
import jax
import jax.numpy as jnp
from jax.experimental import pallas as pl


def kernel(x, mask, table, gamma, beta):
    raise NotImplementedError("write your pallas kernel here")


# speedup vs baseline: 2.1804x; 2.1804x over previous
import functools

import jax
import jax.numpy as jnp
from jax import lax
from jax.experimental import pallas as pl
from jax.experimental.pallas import tpu as pltpu
from jax.experimental.pallas import tpu_sc as plsc

VOCAB = 100000
DIM = 128
EPS = 1e-5


_CHUNK = 128


def _sc_gather(x3d, table, n_rows):
    info = plsc.get_sparse_core_info()
    nc, ns = info.num_cores, info.num_subcores
    nw = nc * ns
    rows_per_w = n_rows // nw
    chunks_per_w = rows_per_w // _CHUNK
    idx_rows_per_w = rows_per_w // _CHUNK

    mesh = plsc.VectorSubcoreMesh(core_axis_name="c", subcore_axis_name="s")

    @functools.partial(
        pl.kernel,
        mesh=mesh,
        out_type=jax.ShapeDtypeStruct((n_rows, DIM), jnp.float32),
        scratch_types=[
            pltpu.VMEM((idx_rows_per_w, _CHUNK), jnp.int32),
            pltpu.VMEM((_CHUNK, DIM), jnp.float32),
            pltpu.SemaphoreType.DMA,
        ],
    )
    def gather_kernel(x_hbm, table_hbm, out_hbm, idx_v, rows_v, sem):
        wid = lax.axis_index("s") * nc + lax.axis_index("c")
        base_out_row = wid * rows_per_w
        pltpu.sync_copy(x_hbm.at[wid], idx_v)

        def body(j, carry):
            pltpu.async_copy(table_hbm.at[idx_v.at[j]], rows_v, sem).wait()
            pltpu.sync_copy(
                rows_v, out_hbm.at[pl.ds(base_out_row + j * _CHUNK, _CHUNK)]
            )
            return carry

        lax.fori_loop(0, chunks_per_w, body, 0)

    return gather_kernel(x3d, table)


_SBLK = 1024


def _stats_body(emb_ref, mask_ref, gamma_ref, beta_ref, scale_ref, shift_ref,
                acc_ref, n_steps):
    i = pl.program_id(0)

    @pl.when(i == 0)
    def _():
        acc_ref[...] = jnp.zeros_like(acc_ref)

    mblk = mask_ref[0]
    eblk = emb_ref[...]
    s = lax.dot_general(mblk, eblk, (((1,), (0,)), ((), ())),
                        preferred_element_type=jnp.float32)
    sq = lax.dot_general(mblk, eblk * eblk, (((1,), (0,)), ((), ())),
                         preferred_element_type=jnp.float32)
    acc_ref[0:1, :] += s
    acc_ref[1:2, :] += sq
    acc_ref[2:3, :] += jnp.full((1, DIM), jnp.sum(mblk), jnp.float32)

    @pl.when(i == n_steps - 1)
    def _():
        cnt = jnp.maximum(acc_ref[2:3, :], 1.0)
        mean = acc_ref[0:1, :] / cnt
        var = jnp.maximum(acc_ref[1:2, :] / cnt - mean * mean, 0.0)
        scale = gamma_ref[...] / jnp.sqrt(var + EPS)
        scale_ref[...] = scale
        shift_ref[...] = beta_ref[...] - mean * scale


def _tc_stats(emb, mask_f, gamma, beta, n_rows):
    n_steps = n_rows // _SBLK
    mask3d = mask_f.reshape(n_steps, 1, _SBLK)
    return pl.pallas_call(
        functools.partial(_stats_body, n_steps=n_steps),
        grid=(n_steps,),
        in_specs=[
            pl.BlockSpec((_SBLK, DIM), lambda i: (i, 0)),
            pl.BlockSpec((1, 1, _SBLK), lambda i: (i, 0, 0)),
            pl.BlockSpec((1, DIM), lambda i: (0, 0)),
            pl.BlockSpec((1, DIM), lambda i: (0, 0)),
        ],
        out_specs=[
            pl.BlockSpec((1, DIM), lambda i: (0, 0)),
            pl.BlockSpec((1, DIM), lambda i: (0, 0)),
        ],
        out_shape=[
            jax.ShapeDtypeStruct((1, DIM), jnp.float32),
            jax.ShapeDtypeStruct((1, DIM), jnp.float32),
        ],
        scratch_shapes=[pltpu.VMEM((3, DIM), jnp.float32)],
    )(emb, mask3d, gamma.reshape(1, DIM), beta.reshape(1, DIM))


_NBLK = 1024


def _norm_body(emb_ref, scale_ref, shift_ref, out_ref):
    out_ref[...] = emb_ref[...] * scale_ref[...] + shift_ref[...]


def _tc_normalize(emb, scale, shift, n_rows):
    return pl.pallas_call(
        _norm_body,
        grid=(n_rows // _NBLK,),
        in_specs=[
            pl.BlockSpec((_NBLK, DIM), lambda i: (i, 0)),
            pl.BlockSpec((1, DIM), lambda i: (0, 0)),
            pl.BlockSpec((1, DIM), lambda i: (0, 0)),
        ],
        out_specs=pl.BlockSpec((_NBLK, DIM), lambda i: (i, 0)),
        out_shape=jax.ShapeDtypeStruct((n_rows, DIM), jnp.float32),
    )(emb, scale, shift)


def kernel(x, mask, table, gamma, beta):
    b, l = x.shape
    n_rows = b * l
    x3d = x.reshape(32, n_rows // (32 * _CHUNK), _CHUNK).astype(jnp.int32)
    emb = _sc_gather(x3d, table, n_rows)
    mask_f = mask.reshape(-1).astype(jnp.float32)
    scale, shift = _tc_stats(emb, mask_f, gamma, beta, n_rows)
    out = _tc_normalize(emb, scale, shift, n_rows)
    return out.reshape(b, l, DIM)


# speedup vs baseline: 2.3905x; 1.0964x over previous
import functools

import jax
import jax.numpy as jnp
from jax import lax
from jax.experimental import pallas as pl
from jax.experimental.pallas import tpu as pltpu
from jax.experimental.pallas import tpu_sc as plsc

VOCAB = 100000
DIM = 128
EPS = 1e-5


_CHUNK = 128


def _sc_gather(x3d, table, n_rows):
    info = plsc.get_sparse_core_info()
    nc, ns = info.num_cores, info.num_subcores
    nw = nc * ns
    rows_per_w = n_rows // nw
    chunks_per_w = rows_per_w // _CHUNK
    idx_rows_per_w = rows_per_w // _CHUNK

    mesh = plsc.VectorSubcoreMesh(core_axis_name="c", subcore_axis_name="s")

    @functools.partial(
        pl.kernel,
        mesh=mesh,
        out_type=jax.ShapeDtypeStruct((n_rows, DIM), jnp.float32),
        scratch_types=[
            pltpu.VMEM((idx_rows_per_w, _CHUNK), jnp.int32),
            pltpu.VMEM((_CHUNK, DIM), jnp.float32),
            pltpu.VMEM((_CHUNK, DIM), jnp.float32),
            pltpu.VMEM((_CHUNK, DIM), jnp.float32),
            pltpu.VMEM((_CHUNK, DIM), jnp.float32),
            pltpu.SemaphoreType.DMA,
            pltpu.SemaphoreType.DMA,
            pltpu.SemaphoreType.DMA,
            pltpu.SemaphoreType.DMA,
            pltpu.SemaphoreType.DMA,
            pltpu.SemaphoreType.DMA,
            pltpu.SemaphoreType.DMA,
            pltpu.SemaphoreType.DMA,
        ],
    )
    def gather_kernel(x_hbm, table_hbm, out_hbm, idx_v,
                      b0, b1, b2, b3, sg0, sg1, sg2, sg3, ss0, ss1, ss2, ss3):
        wid = lax.axis_index("s") * nc + lax.axis_index("c")
        base_out_row = wid * rows_per_w
        pltpu.sync_copy(x_hbm.at[wid], idx_v)

        bufs = (b0, b1, b2, b3)
        sgs = (sg0, sg1, sg2, sg3)
        sss = (ss0, ss1, ss2, ss3)

        for k in range(3):
            pltpu.async_copy(table_hbm.at[idx_v.at[k]], bufs[k], sgs[k])

        def body(j, carry):
            for k in range(4):

                @pl.when(j % 4 == k)
                def _():
                    m = (k + 3) % 4
                    pltpu.make_async_copy(
                        table_hbm.at[idx_v.at[j]], bufs[k], sgs[k]
                    ).wait()
                    pltpu.async_copy(
                        bufs[k],
                        out_hbm.at[pl.ds(base_out_row + j * _CHUNK, _CHUNK)],
                        sss[k],
                    )
                    @pl.when(jnp.logical_and(j >= 1, j + 3 < chunks_per_w))
                    def _():
                        pltpu.make_async_copy(
                            bufs[m],
                            out_hbm.at[pl.ds(base_out_row, _CHUNK)],
                            sss[m],
                        ).wait()

                    @pl.when(j + 3 < chunks_per_w)
                    def _():
                        pltpu.async_copy(
                            table_hbm.at[idx_v.at[j + 3]], bufs[m], sgs[m]
                        )

            return carry

        lax.fori_loop(0, chunks_per_w, body, 0)

        for k in range(4):
            pltpu.make_async_copy(
                bufs[k], out_hbm.at[pl.ds(base_out_row, _CHUNK)], sss[k]
            ).wait()

    return gather_kernel(x3d, table)


_SBLK = 1024


def _stats_body(emb_ref, mask_ref, gamma_ref, beta_ref, scale_ref, shift_ref,
                acc_ref, n_steps):
    i = pl.program_id(0)

    @pl.when(i == 0)
    def _():
        acc_ref[...] = jnp.zeros_like(acc_ref)

    mblk = mask_ref[0]
    eblk = emb_ref[...]
    s = lax.dot_general(mblk, eblk, (((1,), (0,)), ((), ())),
                        preferred_element_type=jnp.float32)
    sq = lax.dot_general(mblk, eblk * eblk, (((1,), (0,)), ((), ())),
                         preferred_element_type=jnp.float32)
    acc_ref[0:1, :] += s
    acc_ref[1:2, :] += sq
    acc_ref[2:3, :] += jnp.full((1, DIM), jnp.sum(mblk), jnp.float32)

    @pl.when(i == n_steps - 1)
    def _():
        cnt = jnp.maximum(acc_ref[2:3, :], 1.0)
        mean = acc_ref[0:1, :] / cnt
        var = jnp.maximum(acc_ref[1:2, :] / cnt - mean * mean, 0.0)
        scale = gamma_ref[...] / jnp.sqrt(var + EPS)
        scale_ref[...] = scale
        shift_ref[...] = beta_ref[...] - mean * scale


def _tc_stats(emb, mask_f, gamma, beta, n_rows):
    n_steps = n_rows // _SBLK
    mask3d = mask_f.reshape(n_steps, 1, _SBLK)
    return pl.pallas_call(
        functools.partial(_stats_body, n_steps=n_steps),
        grid=(n_steps,),
        in_specs=[
            pl.BlockSpec((_SBLK, DIM), lambda i: (i, 0)),
            pl.BlockSpec((1, 1, _SBLK), lambda i: (i, 0, 0)),
            pl.BlockSpec((1, DIM), lambda i: (0, 0)),
            pl.BlockSpec((1, DIM), lambda i: (0, 0)),
        ],
        out_specs=[
            pl.BlockSpec((1, DIM), lambda i: (0, 0)),
            pl.BlockSpec((1, DIM), lambda i: (0, 0)),
        ],
        out_shape=[
            jax.ShapeDtypeStruct((1, DIM), jnp.float32),
            jax.ShapeDtypeStruct((1, DIM), jnp.float32),
        ],
        scratch_shapes=[pltpu.VMEM((3, DIM), jnp.float32)],
    )(emb, mask3d, gamma.reshape(1, DIM), beta.reshape(1, DIM))


_NBLK = 1024


def _norm_body(emb_ref, scale_ref, shift_ref, out_ref):
    out_ref[...] = emb_ref[...] * scale_ref[...] + shift_ref[...]


def _tc_normalize(emb, scale, shift, n_rows):
    return pl.pallas_call(
        _norm_body,
        grid=(n_rows // _NBLK,),
        in_specs=[
            pl.BlockSpec((_NBLK, DIM), lambda i: (i, 0)),
            pl.BlockSpec((1, DIM), lambda i: (0, 0)),
            pl.BlockSpec((1, DIM), lambda i: (0, 0)),
        ],
        out_specs=pl.BlockSpec((_NBLK, DIM), lambda i: (i, 0)),
        out_shape=jax.ShapeDtypeStruct((n_rows, DIM), jnp.float32),
    )(emb, scale, shift)


def kernel(x, mask, table, gamma, beta):
    b, l = x.shape
    n_rows = b * l
    x3d = x.reshape(32, n_rows // (32 * _CHUNK), _CHUNK).astype(jnp.int32)
    emb = _sc_gather(x3d, table, n_rows)
    mask_f = mask.reshape(-1).astype(jnp.float32)
    scale, shift = _tc_stats(emb, mask_f, gamma, beta, n_rows)
    out = _tc_normalize(emb, scale, shift, n_rows)
    return out.reshape(b, l, DIM)
